# baseline (speedup 1.0000x reference)
import functools

import jax
import jax.numpy as jnp
from jax import lax
from jax.experimental import pallas as pl
from jax.experimental.pallas import tpu as pltpu
from jax.experimental.pallas import tpu_sc as plsc

S = 1024
RS = 1025
LAM = 1.0

NW = 32
E = 6_400_000
EW = E // NW
ECH = 10000
NCH = EW // ECH
EG = ECH // 16
EU = 16

N = 100_000
XBR = 51200
XNP = 102_400
NGT = N // 16
NGB = NGT // NW
NXT = NGT - NGB * NW
NWN = (NGB + 1) * 16


def _sc_body(me_hbm, pq_hbm, tq_hbm, mn_hbm, sqx_hbm,
             outq_hbm, outx_hbm, outc_hbm,
             accq, accx, accc,
             pqb0, tqb0, meb0, pqb1, tqb1, meb1,
             sqxb, mnb, obq, obx, obc,
             sem0, sem1, semn):
    wid = lax.axis_index("s") * 2 + lax.axis_index("c")
    iota = lax.broadcasted_iota(jnp.int32, (16,), 0)
    rb = iota * RS
    zeros = jnp.zeros((16,), jnp.float32)
    ones = jnp.ones((16,), jnp.float32)

    ebufs0 = (pqb0, tqb0, meb0)
    ebufs1 = (pqb1, tqb1, meb1)
    ehbm = (pq_hbm, tq_hbm, me_hbm)

    def issue(c, bufs, sem):
        base = wid * EW + c * ECH
        for h, b in zip(ehbm, bufs):
            pltpu.async_copy(h.at[pl.ds(base, ECH)], b, sem)

    def wait_slot(bufs, sem):
        for h, b in zip(ehbm, bufs):
            pltpu.make_async_copy(h.at[pl.ds(0, ECH)], b, sem).wait()

    issue(0, ebufs0, sem0)
    issue(1, ebufs1, sem1)
    g0 = NGB * wid + jnp.minimum(wid, NXT)
    nb = g0 * 16
    pltpu.async_copy(mn_hbm.at[pl.ds(nb, NGB * 16)],
                     mnb.at[pl.ds(0, NGB * 16)], semn)
    pltpu.async_copy(sqx_hbm.at[pl.ds(nb, NGB * 16)],
                     sqxb.at[pl.ds(0, NGB * 16)], semn)

    @plsc.parallel_loop(0, RS, unroll=8)
    def zacc(i):
        o = i * 16
        accq[pl.ds(o, 16)] = zeros
        accx[pl.ds(o, 16)] = zeros
        accc[pl.ds(o, 16)] = zeros

    @pl.when(wid < NXT)
    def _():
        pltpu.sync_copy(mn_hbm.at[pl.ds(nb + NGB * 16, 16)],
                        mnb.at[pl.ds(NGB * 16, 16)])
        pltpu.sync_copy(sqx_hbm.at[pl.ds(nb + NGB * 16, 16)],
                        sqxb.at[pl.ds(NGB * 16, 16)])
    pltpu.make_async_copy(mn_hbm.at[pl.ds(0, NGB * 16)],
                          mnb.at[pl.ds(0, NGB * 16)], semn).wait()
    pltpu.make_async_copy(sqx_hbm.at[pl.ds(0, NGB * 16)],
                          sqxb.at[pl.ds(0, NGB * 16)], semn).wait()

    def ngrp(g):
        sl = pl.ds(g * 16, 16)
        ids = mnb[sl]
        plsc.addupdate_scatter(accc, [rb + ids], ones)
        plsc.addupdate_scatter(accx, [rb + ids], sqxb[sl])

    @plsc.parallel_loop(0, NGB, unroll=8)
    def ngrp_loop(g):
        ngrp(g)

    @pl.when(wid < NXT)
    def _():
        ngrp(NGB)

    def compute(bufs):
        pqb, tqb, meb = bufs

        @plsc.parallel_loop(0, EG, unroll=EU)
        def grp(g):
            sl = pl.ds(g * 16, 16)
            d = pqb[sl] - tqb[sl]
            plsc.addupdate_scatter(accq, [rb + meb[sl]], d * d)

    def pipe(k, carry):
        c0 = 2 * k
        wait_slot(ebufs0, sem0)
        compute(ebufs0)

        @pl.when(c0 + 2 < NCH)
        def _():
            issue(c0 + 2, ebufs0, sem0)
        wait_slot(ebufs1, sem1)
        compute(ebufs1)

        @pl.when(c0 + 3 < NCH)
        def _():
            issue(c0 + 3, ebufs1, sem1)
        return carry
    lax.fori_loop(0, NCH // 2, pipe, 0)

    @plsc.parallel_loop(0, S // 16, unroll=2)
    def fold(c):
        sq = zeros
        sx = zeros
        sc = zeros
        for l in range(16):
            sq = sq + accq[pl.ds(l * RS + c * 16, 16)]
            sx = sx + accx[pl.ds(l * RS + c * 16, 16)]
            sc = sc + accc[pl.ds(l * RS + c * 16, 16)]
        obq[pl.ds(c * 16, 16)] = sq
        obx[pl.ds(c * 16, 16)] = sx
        obc[pl.ds(c * 16, 16)] = sc

    pltpu.sync_copy(obq, outq_hbm.at[wid])
    pltpu.sync_copy(obx, outx_hbm.at[wid])
    pltpu.sync_copy(obc, outc_hbm.at[wid])


_sc_call = functools.partial(
    pl.kernel,
    out_type=(
        jax.ShapeDtypeStruct((NW, S), jnp.float32),
        jax.ShapeDtypeStruct((NW, S), jnp.float32),
        jax.ShapeDtypeStruct((NW, S), jnp.float32),
    ),
    mesh=plsc.VectorSubcoreMesh(core_axis_name="c", subcore_axis_name="s"),
    compiler_params=pltpu.CompilerParams(needs_layout_passes=False),
    scratch_types=[
        pltpu.VMEM((16 * RS,), jnp.float32),
        pltpu.VMEM((16 * RS,), jnp.float32),
        pltpu.VMEM((16 * RS,), jnp.float32),
        pltpu.VMEM((ECH,), jnp.float32),
        pltpu.VMEM((ECH,), jnp.float32),
        pltpu.VMEM((ECH,), jnp.int32),
        pltpu.VMEM((ECH,), jnp.float32),
        pltpu.VMEM((ECH,), jnp.float32),
        pltpu.VMEM((ECH,), jnp.int32),
        pltpu.VMEM((NWN,), jnp.float32),
        pltpu.VMEM((NWN,), jnp.int32),
        pltpu.VMEM((S,), jnp.float32),
        pltpu.VMEM((S,), jnp.float32),
        pltpu.VMEM((S,), jnp.float32),
        pltpu.SemaphoreType.DMA,
        pltpu.SemaphoreType.DMA,
        pltpu.SemaphoreType.DMA,
    ],
)(_sc_body)


def _sqx_body(p_ref, t_ref, o_ref):
    d = p_ref[...] - t_ref[...]
    o_ref[...] = jnp.sum(d * d, axis=0)


def _epi_body(q_ref, x_ref, c_ref, o_ref):
    sq = jnp.sum(q_ref[...], axis=0)
    sx = jnp.sum(x_ref[...], axis=0)
    cnt = jnp.sum(c_ref[...], axis=0)
    norm = jnp.sqrt(sq)
    rmsd = jnp.sqrt(sx / jnp.clip(cnt, 1.0))
    val = (jnp.sum(norm) + LAM * jnp.sum(rmsd)) / S
    o_ref[...] = jnp.full((1, 1), val, jnp.float32)


def kernel(pred_x, pred_q, true_x, true_q, merge_edge, merge_node):
    sqx = pl.pallas_call(
        _sqx_body,
        grid=(XNP // XBR,),
        in_specs=[
            pl.BlockSpec((3, XBR), lambda i: (0, i)),
            pl.BlockSpec((3, XBR), lambda i: (0, i)),
        ],
        out_specs=pl.BlockSpec((XBR,), lambda i: (i,)),
        out_shape=jax.ShapeDtypeStruct((XNP,), jnp.float32),
    )(pred_x.T, true_x.T)

    outq, outx, outc = _sc_call(merge_edge, pred_q, true_q, merge_node, sqx)

    loss = pl.pallas_call(
        _epi_body,
        out_shape=jax.ShapeDtypeStruct((1, 1), jnp.float32),
    )(outq, outx, outc)
    return loss[0, 0]
